# baseline (speedup 1.0000x reference)
import jax
import jax.numpy as jnp
from jax.experimental import pallas as pl


def _pos_body(col_ref, row_ref, out_ref):
    d, hw = out_ref.shape[1] // 2, out_ref.shape[2]
    w = col_ref.shape[0]
    h = row_ref.shape[0]
    col_t = col_ref[...].T
    row_t = row_ref[...].T
    col_b = jnp.broadcast_to(col_t[:, None, :], (d, h, w)).reshape(d, hw)
    row_b = jnp.broadcast_to(row_t[:, :, None], (d, h, w)).reshape(d, hw)
    out_ref[0, :d, :] = col_b
    out_ref[0, d:, :] = row_b


def kernel(x, row_embed, col_embed):
    b = x.shape[0]
    h, w = x.shape[-2], x.shape[-1]
    d = row_embed.shape[1]
    col = col_embed[:w]
    row = row_embed[:h]
    out = pl.pallas_call(
        _pos_body,
        grid=(b,),
        in_specs=[
            pl.BlockSpec((w, d), lambda i: (0, 0)),
            pl.BlockSpec((h, d), lambda i: (0, 0)),
        ],
        out_specs=pl.BlockSpec((1, 2 * d, h * w), lambda i: (i, 0, 0)),
        out_shape=jax.ShapeDtypeStruct((b, 2 * d, h * w), x.dtype),
    )(col, row)
    return out.reshape(b, 2 * d, h, w)


# speedup vs baseline: 1.4651x; 1.4651x over previous
import jax
import jax.numpy as jnp
from jax.experimental import pallas as pl


def _pos_body(t_ref, out_ref):
    d2, hw = out_ref.shape[1], out_ref.shape[2]
    w2 = t_ref.shape[1]
    w = w2 // 2
    h = hw // w
    k_col = jax.lax.broadcasted_iota(jnp.int32, (w2, hw), 1)
    j_row = jax.lax.broadcasted_iota(jnp.int32, (w2, hw), 0)
    sel_top = (k_col % w) == j_row
    sel_bot = (k_col // w + w) == j_row
    m = (sel_top | sel_bot).astype(jnp.float32)
    out_ref[0, :, :] = jnp.dot(
        t_ref[...], m, preferred_element_type=jnp.float32
    )


def kernel(x, row_embed, col_embed):
    b = x.shape[0]
    h, w = x.shape[-2], x.shape[-1]
    d = row_embed.shape[1]
    col_t = col_embed[:w].T
    row_t = row_embed[:h].T
    z_cw = jnp.zeros((d, h), col_t.dtype)
    z_rh = jnp.zeros((d, w), row_t.dtype)
    t = jnp.concatenate(
        [
            jnp.concatenate([col_t, z_cw], axis=1),
            jnp.concatenate([z_rh, row_t], axis=1),
        ],
        axis=0,
    )
    out = pl.pallas_call(
        _pos_body,
        grid=(b,),
        in_specs=[pl.BlockSpec((2 * d, w + h), lambda i: (0, 0))],
        out_specs=pl.BlockSpec((1, 2 * d, h * w), lambda i: (i, 0, 0)),
        out_shape=jax.ShapeDtypeStruct((b, 2 * d, h * w), x.dtype),
    )(t)
    return out.reshape(b, 2 * d, h, w)
